# Initial kernel scaffold; baseline (speedup 1.0000x reference)
import jax
import jax.numpy as jnp
from jax.experimental import pallas as pl


def kernel(x, emb_tables, lin_w, lin_b):
    raise NotImplementedError("write your pallas kernel here")


# speedup vs baseline: 18.8445x; 18.8445x over previous
import functools
import numpy as np
import jax
import jax.numpy as jnp
from jax import lax
from jax.experimental import pallas as pl
from jax.experimental.pallas import tpu as pltpu
from jax.experimental.pallas import tpu_sc as plsc

_N_INV = 56
_D = 16
_PAD = 9
_CAT_IDX = np.array([1, 2, 5, 6, 7, 8, 10, 11, 12, 13, 14, 15, 16, 17, 18, 19,
                     46, 47, 48, 49, 50, 51, 52, 53, 54, 55], dtype=np.int64)
_CONT_IDX = np.array(sorted(set(range(_N_INV)) - set(_CAT_IDX.tolist())),
                     dtype=np.int64)

_NC = 2
_NS = 16
_NW = _NC * _NS
_BATCH = 16384
_ROWS_PER_W = _BATCH // _NW
_CHUNK = 64
_NCHUNK = _ROWS_PER_W // _CHUNK


def _sc_body(x_hbm, tab_hbm, w_hbm, out_hbm, tab_v, w_v, x_v, out_v):
  wid = lax.axis_index("s") * _NC + lax.axis_index("c")
  base = wid * _ROWS_PER_W
  pltpu.sync_copy(tab_hbm, tab_v)
  pltpu.sync_copy(w_hbm, w_v)
  iota16 = lax.iota(jnp.int32, 16)
  fzero16 = jnp.zeros((16,), jnp.float32)

  def chunk_body(g, _):
    row0 = base + g * _CHUNK
    pltpu.sync_copy(x_hbm.at[pl.ds(row0, _CHUNK), :], x_v)

    def row_body(r, _):
      xrow = [x_v[r, pl.ds(0, 16)], x_v[r, pl.ds(16, 16)],
              x_v[r, pl.ds(32, 16)], x_v[r, pl.ds(40, 16)]]
      for n in range(_N_INV):
        k, lane = (n // 16, n % 16) if n < 48 else (3, n - 40)
        xb = xrow[k][lane] + fzero16
        idxv = xb.astype(jnp.int32)
        pos = idxv * _D + (n * (_PAD * _D)) + iota16
        g16 = plsc.load_gather(tab_v, [pos])
        out_v[r, pl.ds(n * _D, _D)] = g16 + xb * w_v[n]
      return 0

    lax.fori_loop(0, _CHUNK, row_body, 0)
    pltpu.sync_copy(out_v, out_hbm.at[pl.ds(row0, _CHUNK), :])
    return 0

  lax.fori_loop(0, _NCHUNK, chunk_body, 0)


def kernel(x, emb_tables, lin_w, lin_b):
  x = x.astype(jnp.float32)
  tab = jnp.zeros((_N_INV, _PAD, _D), jnp.float32)
  tab = tab.at[jnp.asarray(_CAT_IDX)].set(emb_tables)
  tab = tab.at[jnp.asarray(_CONT_IDX)].set(
      jnp.broadcast_to(lin_b[:, None, :], (len(_CONT_IDX), _PAD, _D)))
  w = jnp.zeros((_N_INV, _D), jnp.float32)
  w = w.at[jnp.asarray(_CONT_IDX)].set(lin_w)
  tab_flat = tab.reshape(_N_INV * _PAD * _D)

  mesh = plsc.VectorSubcoreMesh(core_axis_name="c", subcore_axis_name="s")
  run = functools.partial(
      pl.kernel,
      mesh=mesh,
      out_type=jax.ShapeDtypeStruct((_BATCH, _N_INV * _D), jnp.float32),
      compiler_params=pltpu.CompilerParams(needs_layout_passes=False),
      scratch_types=[
          pltpu.VMEM((_N_INV * _PAD * _D,), jnp.float32),
          pltpu.VMEM((_N_INV, _D), jnp.float32),
          pltpu.VMEM((_CHUNK, _N_INV), jnp.float32),
          pltpu.VMEM((_CHUNK, _N_INV * _D), jnp.float32),
      ],
  )(_sc_body)
  out = run(x, tab_flat, w)
  return out.reshape(_BATCH, _N_INV, _D)


# speedup vs baseline: 30.5985x; 1.6237x over previous
import functools
import numpy as np
import jax
import jax.numpy as jnp
from jax import lax
from jax.experimental import pallas as pl
from jax.experimental.pallas import tpu as pltpu
from jax.experimental.pallas import tpu_sc as plsc

_N_INV = 56
_D = 16
_PAD = 9
_CAT_IDX = np.array([1, 2, 5, 6, 7, 8, 10, 11, 12, 13, 14, 15, 16, 17, 18, 19,
                     46, 47, 48, 49, 50, 51, 52, 53, 54, 55], dtype=np.int64)
_CONT_IDX = np.array(sorted(set(range(_N_INV)) - set(_CAT_IDX.tolist())),
                     dtype=np.int64)

_NC = 2
_NS = 16
_NW = _NC * _NS
_BATCH = 16384
_ROWS_PER_W = _BATCH // _NW
_CHUNK = 64
_NCHUNK = _ROWS_PER_W // _CHUNK


def _sc_body(x_hbm, tab_hbm, w_hbm, out_hbm, tab_v, w_v, x_v, out_v):
  wid = lax.axis_index("s") * _NC + lax.axis_index("c")
  base = wid * _ROWS_PER_W
  pltpu.sync_copy(tab_hbm, tab_v)
  pltpu.sync_copy(w_hbm, w_v)
  iota16 = lax.iota(jnp.int32, 16)
  fzero16 = jnp.zeros((16,), jnp.float32)
  nbase = iota16 * (_PAD * _D)
  colbase = [nbase + 16 * k * (_PAD * _D) for k in range(3)]
  colbase.append(nbase + 40 * (_PAD * _D))

  def chunk_body(g, _):
    row0 = base + g * _CHUNK
    pltpu.sync_copy(x_hbm.at[pl.ds(row0, _CHUNK), :], x_v)

    @plsc.parallel_loop(0, _CHUNK, unroll=4)
    def _(r):
      xrow = [x_v[r, pl.ds(0, 16)], x_v[r, pl.ds(16, 16)],
              x_v[r, pl.ds(32, 16)], x_v[r, pl.ds(40, 16)]]
      posrow = [xrow[k].astype(jnp.int32) * _D + colbase[k] for k in range(4)]
      for n in _CAT_IDX.tolist():
        k, lane = (n // 16, n % 16) if n < 48 else (3, n - 40)
        out_v[r, pl.ds(n * _D, _D)] = tab_v[pl.ds(posrow[k][lane], _D)]
      for n in _CONT_IDX.tolist():
        k, lane = (n // 16, n % 16) if n < 48 else (3, n - 40)
        xb = xrow[k][lane] + fzero16
        b16 = tab_v[pl.ds(n * (_PAD * _D), _D)]
        out_v[r, pl.ds(n * _D, _D)] = b16 + xb * w_v[n]

    pltpu.sync_copy(out_v, out_hbm.at[pl.ds(row0, _CHUNK), :])
    return 0

  lax.fori_loop(0, _NCHUNK, chunk_body, 0)


def kernel(x, emb_tables, lin_w, lin_b):
  x = x.astype(jnp.float32)
  tab = jnp.zeros((_N_INV, _PAD, _D), jnp.float32)
  tab = tab.at[jnp.asarray(_CAT_IDX)].set(emb_tables)
  tab = tab.at[jnp.asarray(_CONT_IDX)].set(
      jnp.broadcast_to(lin_b[:, None, :], (len(_CONT_IDX), _PAD, _D)))
  w = jnp.zeros((_N_INV, _D), jnp.float32)
  w = w.at[jnp.asarray(_CONT_IDX)].set(lin_w)
  tab_flat = tab.reshape(_N_INV * _PAD * _D)

  mesh = plsc.VectorSubcoreMesh(core_axis_name="c", subcore_axis_name="s")
  run = functools.partial(
      pl.kernel,
      mesh=mesh,
      out_type=jax.ShapeDtypeStruct((_BATCH, _N_INV * _D), jnp.float32),
      compiler_params=pltpu.CompilerParams(needs_layout_passes=False),
      scratch_types=[
          pltpu.VMEM((_N_INV * _PAD * _D,), jnp.float32),
          pltpu.VMEM((_N_INV, _D), jnp.float32),
          pltpu.VMEM((_CHUNK, _N_INV), jnp.float32),
          pltpu.VMEM((_CHUNK, _N_INV * _D), jnp.float32),
      ],
  )(_sc_body)
  out = run(x, tab_flat, w)
  return out.reshape(_BATCH, _N_INV, _D)
